# baseline (speedup 1.0000x reference)
import functools

import jax
import jax.numpy as jnp
from jax import lax
from jax.experimental import pallas as pl
from jax.experimental.pallas import tpu as pltpu
from jax.experimental.pallas import tpu_sc as plsc

NUM_CLASS = 100000
B = 1024
CTX_DIM = 512
N_CLS_CTX = 4
SEQ_LEN = 77
D = N_CLS_CTX * CTX_DIM

_BB = 8


def _sc_gather(label, table):
    info = plsc.get_sparse_core_info()
    nw = info.num_cores * info.num_subcores
    b_per_w = B // nw
    mesh = plsc.VectorSubcoreMesh(core_axis_name="c", subcore_axis_name="s")

    @functools.partial(
        pl.kernel,
        mesh=mesh,
        out_type=jax.ShapeDtypeStruct((B, D), jnp.float32),
        scratch_types=[
            pltpu.VMEM((b_per_w,), jnp.int32),
            pltpu.VMEM((b_per_w, D), jnp.float32),
            pltpu.SemaphoreType.DMA,
        ],
    )
    def gather_kernel(idx_hbm, table_hbm, out_hbm, idx_v, rows_v, sem):
        wid = lax.axis_index("s") * info.num_cores + lax.axis_index("c")
        base = wid * b_per_w
        pltpu.sync_copy(idx_hbm.at[pl.ds(base, b_per_w)], idx_v)
        pltpu.async_copy(table_hbm.at[idx_v], rows_v, sem).wait()
        pltpu.sync_copy(rows_v, out_hbm.at[pl.ds(base, b_per_w)])

    return gather_kernel(label, table)


def _tc_assemble_body(cls_ref, pre_ref, mid_ref, sp_ref, suf_ref, out_ref):
    bb = out_ref.shape[0]

    def bcast(ref):
        return jnp.broadcast_to(ref[...][None], (bb,) + ref.shape)

    out_ref[...] = jnp.concatenate(
        [bcast(pre_ref), cls_ref[...], bcast(mid_ref), bcast(sp_ref),
         bcast(suf_ref)],
        axis=1,
    )


def kernel(label, cls_ctx, token_prefix, token_middle, token_suffix,
           suffix_prompt):
    table = cls_ctx.reshape(NUM_CLASS, D)
    cls_flat = _sc_gather(label.astype(jnp.int32), table)
    cls = cls_flat.reshape(B, N_CLS_CTX, CTX_DIM)

    pre = token_prefix[0]
    mid = token_middle[0]
    sp = suffix_prompt[0]
    suf = token_suffix[0]
    suffix_len = suf.shape[0]

    grid = (B // _BB,)
    out = pl.pallas_call(
        _tc_assemble_body,
        grid=grid,
        in_specs=[
            pl.BlockSpec((_BB, N_CLS_CTX, CTX_DIM), lambda i: (i, 0, 0)),
            pl.BlockSpec((5, CTX_DIM), lambda i: (0, 0)),
            pl.BlockSpec((2, CTX_DIM), lambda i: (0, 0)),
            pl.BlockSpec((N_CLS_CTX, CTX_DIM), lambda i: (0, 0)),
            pl.BlockSpec((suffix_len, CTX_DIM), lambda i: (0, 0)),
        ],
        out_specs=pl.BlockSpec((_BB, SEQ_LEN, CTX_DIM), lambda i: (i, 0, 0)),
        out_shape=jax.ShapeDtypeStruct((B, SEQ_LEN, CTX_DIM), jnp.float32),
    )(cls, pre, mid, sp, suf)
    return out


# speedup vs baseline: 4.1600x; 4.1600x over previous
import functools

import jax
import jax.numpy as jnp
from jax import lax
from jax.experimental import pallas as pl
from jax.experimental.pallas import tpu as pltpu
from jax.experimental.pallas import tpu_sc as plsc

NUM_CLASS = 100000
B = 1024
CTX_DIM = 512
N_CLS_CTX = 4
SEQ_LEN = 77
D = N_CLS_CTX * CTX_DIM

_BB = 16


def _sc_gather(label, table):
    info = plsc.get_sparse_core_info()
    nw = info.num_cores * info.num_subcores
    b_per_w = B // nw
    mesh = plsc.VectorSubcoreMesh(core_axis_name="c", subcore_axis_name="s")

    @functools.partial(
        pl.kernel,
        mesh=mesh,
        out_type=jax.ShapeDtypeStruct((B, N_CLS_CTX, CTX_DIM), jnp.float32),
        scratch_types=[
            pltpu.VMEM((b_per_w,), jnp.int32),
            pltpu.VMEM((b_per_w, N_CLS_CTX, CTX_DIM), jnp.float32),
            pltpu.SemaphoreType.DMA,
        ],
    )
    def gather_kernel(idx_hbm, table_hbm, out_hbm, idx_v, rows_v, sem):
        wid = lax.axis_index("s") * info.num_cores + lax.axis_index("c")
        base = wid * b_per_w
        pltpu.sync_copy(idx_hbm.at[pl.ds(base, b_per_w)], idx_v)
        pltpu.async_copy(table_hbm.at[idx_v], rows_v, sem).wait()
        pltpu.sync_copy(rows_v, out_hbm.at[pl.ds(base, b_per_w)])

    return gather_kernel(label, table)


def _tc_assemble_body(cls_ref, pre_ref, mid_ref, sp_ref, suf_ref, out_ref):
    bb = out_ref.shape[0]

    def bcast(ref):
        return jnp.broadcast_to(ref[...][None], (bb,) + ref.shape)

    out_ref[...] = jnp.concatenate(
        [bcast(pre_ref), cls_ref[...], bcast(mid_ref), bcast(sp_ref),
         bcast(suf_ref)],
        axis=1,
    )


def kernel(label, cls_ctx, token_prefix, token_middle, token_suffix,
           suffix_prompt):
    cls = _sc_gather(label.astype(jnp.int32), cls_ctx)

    pre = token_prefix[0]
    mid = token_middle[0]
    sp = suffix_prompt[0]
    suf = token_suffix[0]
    suffix_len = suf.shape[0]

    grid = (B // _BB,)
    out = pl.pallas_call(
        _tc_assemble_body,
        grid=grid,
        in_specs=[
            pl.BlockSpec((_BB, N_CLS_CTX, CTX_DIM), lambda i: (i, 0, 0)),
            pl.BlockSpec((5, CTX_DIM), lambda i: (0, 0)),
            pl.BlockSpec((2, CTX_DIM), lambda i: (0, 0)),
            pl.BlockSpec((N_CLS_CTX, CTX_DIM), lambda i: (0, 0)),
            pl.BlockSpec((suffix_len, CTX_DIM), lambda i: (0, 0)),
        ],
        out_specs=pl.BlockSpec((_BB, SEQ_LEN, CTX_DIM), lambda i: (i, 0, 0)),
        out_shape=jax.ShapeDtypeStruct((B, SEQ_LEN, CTX_DIM), jnp.float32),
    )(cls, pre, mid, sp, suf)
    return out
